# baseline (speedup 1.0000x reference)
import functools

import jax
import jax.numpy as jnp
import numpy as np
from jax import lax
from jax.experimental import pallas as pl
from jax.experimental.pallas import tpu as pltpu
from jax.experimental.pallas import tpu_sc as plsc

B = 16384
NUM_COLS = 13
CAT_COLS = 26
VOCAB = 100000
C = 32
L = 16
NC, NS = 2, 16
NW = NC * NS
BPW = B // NW
IDX_PER_W = BPW * CAT_COLS
GPW = BPW // L
GIDX = CAT_COLS * L
IPC = 128
CPS = 13
GPSG = 4
NSG = GPW // GPSG
NBUF = 2

NROWS4 = CAT_COLS * VOCAB // 4
FBLK = 5000


def _fold_body(x_ref, w_ref, o_ref):
    o_ref[...] = jnp.dot(x_ref[...], w_ref[...],
                         preferred_element_type=jnp.float32,
                         precision=lax.Precision.HIGHEST)


def _sc_body(t_hbm, xcat_hbm, xnum_hbm, pat_hbm, par_hbm, out_hbm,
             idx_v, xnum_v, pat_v, par_v, buf, out_v, *sems):
    wid = lax.axis_index("s") * NC + lax.axis_index("c")
    base = wid * BPW

    pltpu.sync_copy(xcat_hbm.at[pl.ds(wid * IDX_PER_W, IDX_PER_W)], idx_v)
    pltpu.sync_copy(
        xnum_hbm.at[pl.ds(wid * BPW * NUM_COLS, BPW * NUM_COLS)], xnum_v)
    pltpu.sync_copy(pat_hbm, pat_v)
    pltpu.sync_copy(par_hbm, par_v)

    pats = [pat_v[pl.ds(k * L, L)] for k in range(CAT_COLS)]

    @pl.loop(0, GPW)
    def _(g):
        gb = g * GIDX
        for k in range(CAT_COLS):
            s = gb + k * L
            idx_v[pl.ds(s, L)] = idx_v[pl.ds(s, L)] + pats[k]

    scls = [par_v[j, 0:L] for j in range(NUM_COLS)]
    constv = par_v[NUM_COLS, 0:L]

    def sg_copy(sg, j, slot):
        return pltpu.make_async_copy(
            t_hbm.at[idx_v.at[pl.ds((sg * CPS + j) * IPC, IPC)]],
            buf.at[slot, pl.ds(j * IPC, IPC)],
            sems[slot])

    for sg in range(NBUF - 1):
        for j in range(CPS):
            sg_copy(sg, j, sg).start()

    @pl.loop(0, NSG, step=NBUF)
    def _(sg0):
        for s in range(NBUF):
            sg = sg0 + s

            @pl.when(sg + NBUF - 1 < NSG)
            def _():
                for j in range(CPS):
                    sg_copy(sg + NBUF - 1, j, (s + NBUF - 1) % NBUF).start()

            for j in range(CPS):
                sg_copy(sg, j, s).wait()

            for gl in range(GPSG):
                g = sg * GPSG + gl
                bb = gl * GIDX
                acc = constv + buf[s, pl.ds(bb, L)]
                for k in range(1, CAT_COLS):
                    acc = acc + buf[s, pl.ds(bb + k * L, L)]
                nb = g * NUM_COLS * L
                for j in range(NUM_COLS):
                    acc = acc + xnum_v[pl.ds(nb + j * L, L)] * scls[j]
                out_v[pl.ds(g * L, L)] = acc

    pltpu.sync_copy(out_v, out_hbm.at[pl.ds(base, BPW)])


@jax.jit
def _run(emb4, w4, xcat_r, xnum_r, pat, par):
    t4 = pl.pallas_call(
        _fold_body,
        grid=(NROWS4 // FBLK,),
        in_specs=[
            pl.BlockSpec((FBLK, 128), lambda i: (i, 0)),
            pl.BlockSpec((128, 4), lambda i: (0, 0)),
        ],
        out_specs=pl.BlockSpec((FBLK, 4), lambda i: (i, 0)),
        out_shape=jax.ShapeDtypeStruct((NROWS4, 4), jnp.float32),
    )(emb4, w4)
    t = t4.reshape(-1)

    mesh = plsc.VectorSubcoreMesh(core_axis_name="c", subcore_axis_name="s")
    f = functools.partial(
        pl.kernel,
        out_type=jax.ShapeDtypeStruct((B,), jnp.float32),
        mesh=mesh,
        compiler_params=pltpu.CompilerParams(
            needs_layout_passes=False, use_tc_tiling_on_sc=False),
        scratch_types=[
            pltpu.VMEM((IDX_PER_W,), jnp.int32),
            pltpu.VMEM((BPW * NUM_COLS,), jnp.float32),
            pltpu.VMEM((GIDX,), jnp.int32),
            pltpu.VMEM((NUM_COLS + 1, L), jnp.float32),
            pltpu.VMEM((NBUF, CPS * IPC), jnp.float32),
            pltpu.VMEM((BPW,), jnp.float32),
        ] + [pltpu.SemaphoreType.DMA] * NBUF,
    )(_sc_body)
    return f(t, xcat_r, xnum_r, pat, par)


def kernel(x_num, x_cat, col_mean, col_std, W_num, b_num, emb, W_out, b_out):
    ncols = NUM_COLS + CAT_COLS
    v = W_out[:, 0]
    u = W_num @ v
    scl = u / col_std
    const = (jnp.sum(b_num @ v) - jnp.sum(col_mean * scl)) / ncols + b_out[0]

    w4 = (jnp.eye(4, dtype=jnp.float32)[:, None, :]
          * (v / ncols)[None, :, None]).reshape(128, 4)
    emb4 = emb.reshape(NROWS4, 128)

    par = jnp.concatenate([
        jnp.broadcast_to((scl / ncols)[:, None], (NUM_COLS, L)),
        jnp.broadcast_to(jnp.reshape(const, (1, 1)), (1, L)),
    ])
    pat = jnp.asarray(np.repeat(np.arange(CAT_COLS) * VOCAB, L),
                      dtype=jnp.int32)

    xcat_r = x_cat.reshape(B // L, L, CAT_COLS).transpose(0, 2, 1).reshape(-1)
    xnum_r = x_num.reshape(B // L, L, NUM_COLS).transpose(0, 2, 1).reshape(-1)
    return _run(emb4, w4, xcat_r, xnum_r, pat, par)


# speedup vs baseline: 1.0810x; 1.0810x over previous
import functools

import jax
import jax.numpy as jnp
import numpy as np
from jax import lax
from jax.experimental import pallas as pl
from jax.experimental.pallas import tpu as pltpu
from jax.experimental.pallas import tpu_sc as plsc

B = 16384
NUM_COLS = 13
CAT_COLS = 26
VOCAB = 100000
C = 32
L = 16
NC, NS = 2, 16
NW = NC * NS
BPW = B // NW

M = 10
NA = CAT_COLS - M

IPC_A = 128
RPC_A = IPC_A // NA
NCH_A = BPW // RPC_A
IDXA_PER_W = BPW * NA
NBUF_A = 4

IDXB_PER_W = BPW * M
NCH_B = IDXB_PER_W // 128
PAT_B = 80
WAVE_B = 8

FROWS4 = M * VOCAB // 4
FBLK = 10000


def _fold_body(x_ref, w_ref, o_ref):
    o_ref[...] = jnp.dot(x_ref[...], w_ref[...],
                         preferred_element_type=jnp.float32,
                         precision=lax.Precision.HIGHEST)


def _sc_a_body(emb_hbm, xcat_hbm, xnum_hbm, pat_hbm, par_hbm, out_hbm,
               idx_v, xnum_v, pat_v, par_v, buf, tmat_v, out_v, *sems):
    wid = lax.axis_index("s") * NC + lax.axis_index("c")
    base = wid * BPW

    pltpu.sync_copy(xcat_hbm.at[pl.ds(wid * IDXA_PER_W, IDXA_PER_W)], idx_v)
    pltpu.sync_copy(xnum_hbm.at[pl.ds(base * L, BPW * L)], xnum_v)
    pltpu.sync_copy(pat_hbm, pat_v)
    pltpu.sync_copy(par_hbm, par_v)

    patv = pat_v[0:L]

    @pl.loop(0, BPW)
    def _(r):
        idx_v[pl.ds(r * NA, NA)] = idx_v[pl.ds(r * NA, NA)] + patv

    vs_lo = par_v[0, 0:L]
    vs_hi = par_v[1, 0:L]
    scl = par_v[2, 0:L]
    rowbase = lax.iota(jnp.int32, L) * L

    def chunk_copy(c, slot):
        return pltpu.make_async_copy(
            emb_hbm.at[idx_v.at[pl.ds(c * IPC_A, IPC_A)]], buf.at[slot],
            sems[slot])

    for c in range(NBUF_A - 1):
        chunk_copy(c, c).start()

    @pl.loop(0, NCH_A, step=NBUF_A)
    def _(c0):
        for h in range(2):
            for hs in range(2):
                s = h * 2 + hs
                c = c0 + s

                @pl.when(c + NBUF_A - 1 < NCH_A)
                def _():
                    chunk_copy(c + NBUF_A - 1, (s + NBUF_A - 1) % NBUF_A
                               ).start()

                chunk_copy(c, s).wait()

                for r in range(RPC_A):
                    rb = r * NA
                    acc0 = buf[s, rb, 0:L]
                    acc1 = buf[s, rb, L:C]
                    for k in range(1, NA):
                        acc0 = acc0 + buf[s, rb + k, 0:L]
                        acc1 = acc1 + buf[s, rb + k, L:C]
                    row = c * RPC_A + r
                    t = (acc0 * vs_lo + acc1 * vs_hi
                         + xnum_v[pl.ds(row * L, L)] * scl)
                    tmat_v[pl.ds((hs * RPC_A + r) * L, L)] = t

            ovec = plsc.load_gather(tmat_v, [rowbase])
            for col in range(1, L):
                ovec = ovec + plsc.load_gather(tmat_v, [rowbase + col])
            out_v[pl.ds(c0 * RPC_A + h * L, L)] = ovec

    pltpu.sync_copy(out_v, out_hbm.at[pl.ds(base, BPW)])


def _sc_b_body(t_hbm, xcat_hbm, part_hbm, pat_hbm, out_hbm,
               idx_v, pat_v, part_v, buf_v, out_v, sem):
    wid = lax.axis_index("s") * NC + lax.axis_index("c")
    base = wid * BPW

    pltpu.sync_copy(xcat_hbm.at[pl.ds(wid * IDXB_PER_W, IDXB_PER_W)], idx_v)
    pltpu.sync_copy(part_hbm.at[pl.ds(base, BPW)], part_v)
    pltpu.sync_copy(pat_hbm, pat_v)

    pats = [pat_v[pl.ds(j * L, L)] for j in range(PAT_B // L)]

    @pl.loop(0, IDXB_PER_W // PAT_B)
    def _(g):
        gb = g * PAT_B
        for j in range(PAT_B // L):
            s = gb + j * L
            idx_v[pl.ds(s, L)] = idx_v[pl.ds(s, L)] + pats[j]

    def chunk_copy(c):
        return pltpu.make_async_copy(
            t_hbm.at[idx_v.at[pl.ds(c * 128, 128)]],
            buf_v.at[pl.ds(c * 128, 128)], sem)

    for w0 in range(0, NCH_B, WAVE_B):
        for c in range(w0, min(w0 + WAVE_B, NCH_B)):
            chunk_copy(c).start()
        for c in range(w0, min(w0 + WAVE_B, NCH_B)):
            chunk_copy(c).wait()

    rowbase10 = lax.iota(jnp.int32, L) * M
    for g in range(BPW // L):
        gb = g * L * M
        acc = part_v[pl.ds(g * L, L)]
        for j in range(M):
            acc = acc + plsc.load_gather(buf_v, [rowbase10 + (gb + j)])
        out_v[pl.ds(g * L, L)] = acc

    pltpu.sync_copy(out_v, out_hbm.at[pl.ds(base, BPW)])


@jax.jit
def _run(emb_flat, emb4m, w4, xcata, xcatb, xnum_pad, pata, patb, par):
    mesh = plsc.VectorSubcoreMesh(core_axis_name="c", subcore_axis_name="s")
    sc_params = pltpu.CompilerParams(
        needs_layout_passes=False, use_tc_tiling_on_sc=False)

    fa = functools.partial(
        pl.kernel,
        out_type=jax.ShapeDtypeStruct((B,), jnp.float32),
        mesh=mesh,
        compiler_params=sc_params,
        scratch_types=[
            pltpu.VMEM((IDXA_PER_W,), jnp.int32),
            pltpu.VMEM((BPW * L,), jnp.float32),
            pltpu.VMEM((L,), jnp.int32),
            pltpu.VMEM((3, L), jnp.float32),
            pltpu.VMEM((NBUF_A, IPC_A, C), jnp.float32),
            pltpu.VMEM((L * L,), jnp.float32),
            pltpu.VMEM((BPW,), jnp.float32),
        ] + [pltpu.SemaphoreType.DMA] * NBUF_A,
    )(_sc_a_body)
    part = fa(emb_flat, xcata, xnum_pad, pata, par)

    t4 = pl.pallas_call(
        _fold_body,
        grid=(FROWS4 // FBLK,),
        in_specs=[
            pl.BlockSpec((FBLK, 128), lambda i: (i, 0)),
            pl.BlockSpec((128, 4), lambda i: (0, 0)),
        ],
        out_specs=pl.BlockSpec((FBLK, 4), lambda i: (i, 0)),
        out_shape=jax.ShapeDtypeStruct((FROWS4, 4), jnp.float32),
    )(emb4m, w4)
    t = t4.reshape(-1)

    fb = functools.partial(
        pl.kernel,
        out_type=jax.ShapeDtypeStruct((B,), jnp.float32),
        mesh=mesh,
        compiler_params=sc_params,
        scratch_types=[
            pltpu.VMEM((IDXB_PER_W,), jnp.int32),
            pltpu.VMEM((PAT_B,), jnp.int32),
            pltpu.VMEM((BPW,), jnp.float32),
            pltpu.VMEM((IDXB_PER_W,), jnp.float32),
            pltpu.VMEM((BPW,), jnp.float32),
            pltpu.SemaphoreType.DMA,
        ],
    )(_sc_b_body)
    return fb(t, xcatb, part, patb)


def kernel(x_num, x_cat, col_mean, col_std, W_num, b_num, emb, W_out, b_out):
    ncols = NUM_COLS + CAT_COLS
    v = W_out[:, 0]
    u = W_num @ v
    scl = u / col_std
    const = (jnp.sum(b_num @ v) - jnp.sum(col_mean * scl)) / ncols + b_out[0]

    scl_full = jnp.concatenate([
        scl / ncols,
        jnp.reshape(const, (1,)),
        jnp.zeros((L - NUM_COLS - 1,), jnp.float32),
    ])
    par = jnp.stack([v[0:L] / ncols, v[L:C] / ncols, scl_full])

    w4 = (jnp.eye(4, dtype=jnp.float32)[:, None, :]
          * (v / ncols)[None, :, None]).reshape(128, 4)

    emb_flat = emb.reshape(CAT_COLS * VOCAB, C)
    emb4m = emb.reshape(CAT_COLS * VOCAB // 4, 128)[:FROWS4]

    pata = jnp.asarray((M + np.arange(L)) * VOCAB, dtype=jnp.int32)
    patb = jnp.asarray((np.arange(PAT_B) % M) * VOCAB, dtype=jnp.int32)

    xcata = x_cat[:, M:].reshape(-1)
    xcatb = x_cat[:, :M].reshape(-1)
    xnum_pad = jnp.concatenate([
        x_num,
        jnp.ones((B, 1), jnp.float32),
        jnp.zeros((B, L - NUM_COLS - 1), jnp.float32),
    ], axis=1).reshape(-1)
    return _run(emb_flat, emb4m, w4, xcata, xcatb, xnum_pad, pata, patb, par)


# speedup vs baseline: 1.6576x; 1.5334x over previous
import functools

import jax
import jax.numpy as jnp
import numpy as np
from jax import lax
from jax.experimental import pallas as pl
from jax.experimental.pallas import tpu as pltpu
from jax.experimental.pallas import tpu_sc as plsc

B = 16384
NUM_COLS = 13
CAT_COLS = 26
VOCAB = 100000
C = 32
L = 16
NC, NS = 2, 16
NW = NC * NS
BPW = B // NW
RPC = 4
IPC = RPC * CAT_COLS
NCHUNK = BPW // RPC
IDX_PER_W = BPW * CAT_COLS
PAT = 208


NBUF = 4


def _sc_body(emb_hbm, xcat_hbm, xnum_hbm, pat_hbm, par_hbm, out_hbm,
             idx_v, xnum_v, pat_v, par_v, buf, tmat_v, out_v, *sems):
    wid = lax.axis_index("s") * NC + lax.axis_index("c")
    base = wid * BPW

    pltpu.sync_copy(xcat_hbm.at[pl.ds(wid * IDX_PER_W, IDX_PER_W)], idx_v)
    pltpu.sync_copy(xnum_hbm.at[pl.ds(base * L, BPW * L)], xnum_v)
    pltpu.sync_copy(pat_hbm, pat_v)
    pltpu.sync_copy(par_hbm, par_v)

    pats = [pat_v[pl.ds(j * L, L)] for j in range(PAT // L)]

    @pl.loop(0, IDX_PER_W // PAT)
    def _(g):
        gb = g * PAT
        for j in range(PAT // L):
            s = gb + j * L
            idx_v[pl.ds(s, L)] = idx_v[pl.ds(s, L)] + pats[j]

    vs_lo = par_v[0, 0:L]
    vs_hi = par_v[1, 0:L]
    scl = par_v[2, 0:L]
    rowbase = lax.iota(jnp.int32, L) * L

    def chunk_copy(c, slot):
        return pltpu.make_async_copy(
            emb_hbm.at[idx_v.at[pl.ds(c * IPC, IPC)]], buf.at[slot],
            sems[slot])

    for c in range(NBUF - 1):
        chunk_copy(c, c).start()

    @pl.loop(0, NCHUNK, step=4)
    def _(c0):
        for s in range(4):
            c = c0 + s
            slot = s % NBUF

            @pl.when(c + NBUF - 1 < NCHUNK)
            def _():
                chunk_copy(c + NBUF - 1, (s + NBUF - 1) % NBUF).start()

            chunk_copy(c, slot).wait()

            for r in range(RPC):
                rb = r * CAT_COLS
                acc0 = buf[slot, rb, 0:L]
                acc1 = buf[slot, rb, L:C]
                for k in range(1, CAT_COLS):
                    acc0 = acc0 + buf[slot, rb + k, 0:L]
                    acc1 = acc1 + buf[slot, rb + k, L:C]
                row = c * RPC + r
                t = (acc0 * vs_lo + acc1 * vs_hi
                     + xnum_v[pl.ds(row * L, L)] * scl)
                tmat_v[pl.ds((s * RPC + r) * L, L)] = t

        ovec = plsc.load_gather(tmat_v, [rowbase])
        for col in range(1, L):
            ovec = ovec + plsc.load_gather(tmat_v, [rowbase + col])
        out_v[pl.ds(c0 * RPC, L)] = ovec

    pltpu.sync_copy(out_v, out_hbm.at[pl.ds(base, BPW)])


@jax.jit
def _run(emb_flat, xcat_flat, xnum_pad, pat, par):
    mesh = plsc.VectorSubcoreMesh(core_axis_name="c", subcore_axis_name="s")
    f = functools.partial(
        pl.kernel,
        out_type=jax.ShapeDtypeStruct((B,), jnp.float32),
        mesh=mesh,
        compiler_params=pltpu.CompilerParams(
            needs_layout_passes=False, use_tc_tiling_on_sc=False),
        scratch_types=[
            pltpu.VMEM((IDX_PER_W,), jnp.int32),
            pltpu.VMEM((BPW * L,), jnp.float32),
            pltpu.VMEM((PAT,), jnp.int32),
            pltpu.VMEM((3, L), jnp.float32),
            pltpu.VMEM((NBUF, IPC, C), jnp.float32),
            pltpu.VMEM((L * L,), jnp.float32),
            pltpu.VMEM((BPW,), jnp.float32),
        ] + [pltpu.SemaphoreType.DMA] * NBUF,
    )(_sc_body)
    return f(emb_flat, xcat_flat, xnum_pad, pat, par)


def kernel(x_num, x_cat, col_mean, col_std, W_num, b_num, emb, W_out, b_out):
    v = W_out[:, 0]
    u = W_num @ v
    scl = u / col_std
    ncols = NUM_COLS + CAT_COLS
    const = (jnp.sum(b_num @ v) - jnp.sum(col_mean * scl)) / ncols + b_out[0]

    scl_full = jnp.concatenate([
        scl / ncols,
        jnp.reshape(const, (1,)),
        jnp.zeros((L - NUM_COLS - 1,), jnp.float32),
    ])
    par = jnp.stack([v[0:L] / ncols, v[L:C] / ncols, scl_full])
    pat = jnp.asarray((np.arange(PAT) % CAT_COLS) * VOCAB, dtype=jnp.int32)

    emb_flat = emb.reshape(CAT_COLS * VOCAB, C)
    xcat_flat = x_cat.reshape(-1)
    xnum_pad = jnp.concatenate([
        x_num,
        jnp.ones((B, 1), jnp.float32),
        jnp.zeros((B, L - NUM_COLS - 1), jnp.float32),
    ], axis=1).reshape(-1)
    return _run(emb_flat, xcat_flat, xnum_pad, pat, par)
